# baseline (speedup 1.0000x reference)
import jax
import jax.numpy as jnp
from jax import lax
from jax.experimental import pallas as pl
from jax.experimental.pallas import tpu as pltpu
from jax.experimental.pallas import tpu_sc as plsc

N = 10000
E = 160000
D = 256

NC = 2
NS = 16
NW = NC * NS
L = 16

N_PAD = 10240
E_PAD = 163840
CHUNK = 128

WIN = N_PAD // NW
TRASHW = WIN
WROWS = WIN + 8

BLK = 2048
NBLK = E_PAD // BLK
SPILL = 8192
BUFN = 10496
TAIL = BUFN - SPILL
LIST_CAP = E_PAD + 128
SHIFT = 9

EPW = E_PAD // NW
DGROUPS = EPW // CHUNK

_MESH = plsc.VectorSubcoreMesh(
    core_axis_name="c", subcore_axis_name="s", num_cores=NC, num_subcores=NS
)
_SC_PARAMS = pltpu.CompilerParams(needs_layout_passes=False)


def _wid():
    return lax.axis_index("s") * NC + lax.axis_index("c")


def _route_body(src_hbm, dst_hbm, lists_hbm, counts_hbm, deg_hbm,
                buf, sidc, dlc, degf, cntv):
    wid = _wid()
    lo = wid * WIN
    base = wid * LIST_CAP
    ones = jnp.ones((L,), jnp.float32)
    zf = jnp.zeros((L,), jnp.float32)

    def zb(i, _):
        degf[pl.ds(i * L, L)] = zf
        return 0

    lax.fori_loop(0, (WIN + L) // L, zb, 0)

    def blk_body(b, carry):
        nm, oo = carry
        off = b * BLK
        pltpu.sync_copy(src_hbm.at[pl.ds(pl.multiple_of(off, 128), BLK)], sidc)
        pltpu.sync_copy(dst_hbm.at[pl.ds(pl.multiple_of(off, 128), BLK)], dlc)

        def g_body(gq, nm):
            svals, cnts = [], []
            for t in range(4):
                g = gq * 4 + t
                sv = sidc[pl.ds(g * L, L)]
                dv = dlc[pl.ds(g * L, L)]
                m = (dv >= lo) & (dv < lo + WIN)
                dl = jnp.where(m, dv - lo, jnp.int32(TRASHW))
                plsc.addupdate_scatter(degf, [dl], ones)
                packed = (sv << SHIFT) | dl
                key = jnp.where(m, jnp.int32(0), jnp.int32(1))
                _, sval = plsc.sort_key_val(key, packed)
                svals.append(sval)
                cnts.append(plsc.all_reduce_population_count(m)[0])
            off = nm
            for t in range(4):
                buf[pl.ds(off, L)] = svals[t]
                off = off + cnts[t]
            return off

        nm = lax.fori_loop(0, BLK // L // 4, g_body, nm)

        do_spill = nm >= SPILL

        @pl.when(do_spill)
        def _():
            pltpu.sync_copy(
                buf.at[pl.ds(0, SPILL)],
                lists_hbm.at[pl.ds(pl.multiple_of(base + oo, 128), SPILL)]
            )

            def mv(q, _):
                buf[pl.ds(q * L, L)] = buf[pl.ds(SPILL + q * L, L)]
                return 0

            lax.fori_loop(0, TAIL // L, mv, 0)

        nm = jnp.where(do_spill, nm - SPILL, nm)
        oo = jnp.where(do_spill, oo + SPILL, oo)
        return (nm, oo)

    nm, oo = lax.fori_loop(0, NBLK, blk_body, (jnp.int32(0), jnp.int32(0)))

    trash16 = jnp.full((L,), TRASHW, jnp.int32)

    def pad_b(j, _):
        buf[pl.ds(nm + j * L, L)] = trash16
        return 0

    lax.fori_loop(0, CHUNK // L, pad_b, 0)

    nflush = (nm + CHUNK - 1) // CHUNK

    def fl(q, _):
        pltpu.sync_copy(
            buf.at[pl.ds(q * CHUNK, CHUNK)],
            lists_hbm.at[pl.ds(pl.multiple_of(base + oo + q * CHUNK, 128), CHUNK)],
        )
        return 0

    lax.fori_loop(0, nflush, fl, 0)

    cntv[pl.ds(0, L)] = jnp.full((L,), 0, jnp.int32) + (oo + nm)
    pltpu.sync_copy(cntv, counts_hbm.at[pl.ds(pl.multiple_of(wid * L, 16), L)])
    pltpu.sync_copy(degf.at[pl.ds(0, WIN)],
                    deg_hbm.at[pl.ds(pl.multiple_of(wid * WIN, 64), WIN)])


_route_call = pl.kernel(
    _route_body,
    out_type=(
        jax.ShapeDtypeStruct((NW * LIST_CAP,), jnp.int32),
        jax.ShapeDtypeStruct((NW * L,), jnp.int32),
        jax.ShapeDtypeStruct((N_PAD,), jnp.float32),
    ),
    mesh=_MESH,
    compiler_params=_SC_PARAMS,
    scratch_types=[
        pltpu.VMEM((BUFN,), jnp.int32),
        pltpu.VMEM((BLK,), jnp.int32),
        pltpu.VMEM((BLK,), jnp.int32),
        pltpu.VMEM((WIN + L,), jnp.float32),
        pltpu.VMEM((L,), jnp.int32),
    ],
)


CH = 64
MASK9 = (1 << SHIFT) - 1


def _segsum_body(u_hbm, lists_hbm, counts_hbm, out_hbm,
                 win, pk, sidc0, sidc1, dlc0, dlc1, stg0, stg1, cntv,
                 sem0, sem1):
    wid = _wid()
    base = wid * LIST_CAP
    zf = jnp.zeros((L,), jnp.float32)
    sidc = (sidc0, sidc1)
    dlc = (dlc0, dlc1)
    stg = (stg0, stg1)
    sem = (sem0, sem1)

    def zb(i, _):
        for j in range(D // L):
            win[i, pl.ds(j * L, L)] = zf
        return 0

    lax.fori_loop(0, WROWS, zb, 0)

    pltpu.sync_copy(counts_hbm.at[pl.ds(pl.multiple_of(wid * L, 16), L)], cntv)
    total = cntv[pl.ds(0, L)][0]
    nch = (total + CH - 1) // CH

    def start(ci, b):
        pltpu.sync_copy(
            lists_hbm.at[pl.ds(pl.multiple_of(base + ci * CH, 64), CH)], pk)
        for j in range(CH // L):
            p = pk[pl.ds(j * L, L)]
            sidc[b][pl.ds(j * L, L)] = p >> SHIFT
            dlc[b][pl.ds(j * L, L)] = p & MASK9
        pltpu.async_copy(u_hbm.at[sidc[b]], stg[b], sem[b])

    @pl.when(nch > 0)
    def _():
        start(0, 0)

    def outer(t, _):
        for b in range(2):
            ci = t * 2 + b

            @pl.when(ci < nch)
            def _(b=b, ci=ci):
                pltpu.make_async_copy(u_hbm.at[sidc[b]], stg[b], sem[b]).wait()

                @pl.when(ci + 1 < nch)
                def _():
                    start(ci + 1, 1 - b)

                def accg(g, _):
                    rv = dlc[b][pl.ds(g * L, L)]
                    for i in range(L):
                        k = g * L + i
                        r = rv[i]
                        for j in range(D // L):
                            plsc.addupdate(
                                win.at[r, pl.ds(j * L, L)],
                                stg[b][k, pl.ds(j * L, L)],
                            )
                    return 0

                lax.fori_loop(0, CH // L, accg, 0)

        return 0

    lax.fori_loop(0, (nch + 1) // 2, outer, 0)

    pltpu.sync_copy(win.at[pl.ds(0, WIN)],
                    out_hbm.at[pl.ds(pl.multiple_of(wid * WIN, 64), WIN)])


_segsum_call = pl.kernel(
    _segsum_body,
    out_type=jax.ShapeDtypeStruct((N_PAD, D), jnp.float32),
    mesh=_MESH,
    compiler_params=_SC_PARAMS,
    scratch_types=[
        pltpu.VMEM((WROWS, D), jnp.float32),
        pltpu.VMEM((CH,), jnp.int32),
        pltpu.VMEM((CH,), jnp.int32),
        pltpu.VMEM((CH,), jnp.int32),
        pltpu.VMEM((CH + L,), jnp.int32),
        pltpu.VMEM((CH + L,), jnp.int32),
        pltpu.VMEM((CH, D), jnp.float32),
        pltpu.VMEM((CH, D), jnp.float32),
        pltpu.VMEM((L,), jnp.int32),
        pltpu.SemaphoreType.DMA,
        pltpu.SemaphoreType.DMA,
    ],
)


DCH = 64
DCHUNKS = EPW // DCH


def _decode_body(z_hbm, src_hbm, dst_hbm, out_hbm,
                 sidx0, sidx1, didx0, didx1, ar0, ar1, br0, br1, stag, outv,
                 sa0, sa1, sb0, sb1):
    c = lax.axis_index("c")
    s = lax.axis_index("s")
    iota = lax.iota(jnp.int32, L)
    sidx = (sidx0, sidx1)
    didx = (didx0, didx1)
    ar = (ar0, ar1)
    br = (br0, br1)
    sa = (sa0, sa1)
    sb = (sb0, sb1)

    K0, K1 = 64, 96
    kcount = jnp.where(c == 0, jnp.int32(K0), jnp.int32(K1))
    cbase = jnp.where(c == 0, s * K0, NS * K0 + s * K1)

    def chunk_off(ci):
        return pl.multiple_of((cbase + ci) * DCH, 64)

    def start(ci, b):
        off = chunk_off(ci)
        pltpu.sync_copy(src_hbm.at[pl.ds(off, DCH)], sidx[b])
        pltpu.sync_copy(dst_hbm.at[pl.ds(off, DCH)], didx[b])
        pltpu.async_copy(z_hbm.at[sidx[b]], ar[b], sa[b])
        pltpu.async_copy(z_hbm.at[didx[b]], br[b], sb[b])

    start(0, 0)

    def outer(t, _):
        for b in range(2):
            ci = t * 2 + b

            @pl.when(ci < kcount)
            def _(b=b, ci=ci):
                pltpu.make_async_copy(z_hbm.at[sidx[b]], ar[b], sa[b]).wait()
                pltpu.make_async_copy(z_hbm.at[didx[b]], br[b], sb[b]).wait()

                @pl.when(ci + 1 < kcount)
                def _(b=b, ci=ci):
                    start(ci + 1, 1 - b)

                def group(g, _, b=b):
                    for i in range(L):
                        r = g * L + i
                        acc0 = ar[b][r, pl.ds(0, L)] * br[b][r, pl.ds(0, L)]
                        acc1 = ar[b][r, pl.ds(L, L)] * br[b][r, pl.ds(L, L)]
                        acc2 = ar[b][r, pl.ds(2 * L, L)] * br[b][r, pl.ds(2 * L, L)]
                        acc3 = ar[b][r, pl.ds(3 * L, L)] * br[b][r, pl.ds(3 * L, L)]
                        for j in range(4, D // L, 4):
                            acc0 = acc0 + ar[b][r, pl.ds(j * L, L)] * br[b][r, pl.ds(j * L, L)]
                            acc1 = acc1 + ar[b][r, pl.ds((j + 1) * L, L)] * br[b][r, pl.ds((j + 1) * L, L)]
                            acc2 = acc2 + ar[b][r, pl.ds((j + 2) * L, L)] * br[b][r, pl.ds((j + 2) * L, L)]
                            acc3 = acc3 + ar[b][r, pl.ds((j + 3) * L, L)] * br[b][r, pl.ds((j + 3) * L, L)]
                        stag[i, pl.ds(0, L)] = (acc0 + acc1) + (acc2 + acc3)
                    out16 = jnp.zeros((L,), jnp.float32)
                    for k in range(L):
                        col = jnp.full((L,), 0, jnp.int32) + k
                        out16 = out16 + plsc.load_gather(stag, [iota, col])
                    outv[pl.ds(g * L, L)] = out16
                    return 0

                lax.fori_loop(0, DCH // L, group, 0)
                pltpu.sync_copy(outv, out_hbm.at[pl.ds(chunk_off(ci), DCH)])

        return 0

    lax.fori_loop(0, (K1 + 1) // 2, outer, 0)


_decode_call = pl.kernel(
    _decode_body,
    out_type=jax.ShapeDtypeStruct((E_PAD,), jnp.float32),
    mesh=_MESH,
    compiler_params=_SC_PARAMS,
    scratch_types=[
        pltpu.VMEM((DCH,), jnp.int32),
        pltpu.VMEM((DCH,), jnp.int32),
        pltpu.VMEM((DCH,), jnp.int32),
        pltpu.VMEM((DCH,), jnp.int32),
        pltpu.VMEM((DCH, D), jnp.float32),
        pltpu.VMEM((DCH, D), jnp.float32),
        pltpu.VMEM((DCH, D), jnp.float32),
        pltpu.VMEM((DCH, D), jnp.float32),
        pltpu.VMEM((L, L), jnp.float32),
        pltpu.VMEM((DCH,), jnp.float32),
        pltpu.SemaphoreType.DMA,
        pltpu.SemaphoreType.DMA,
        pltpu.SemaphoreType.DMA,
        pltpu.SemaphoreType.DMA,
    ],
)


_RB = 1024
_GRID = N_PAD // _RB


def _tc1_body(deg_ref, x_ref, w_ref, b_ref, u_ref, base_ref, dis_ref):
    dis = lax.rsqrt(deg_ref[...] + 1.0)
    h = jnp.dot(x_ref[...], w_ref[...], preferred_element_type=jnp.float32)
    u_ref[...] = dis * h
    base_ref[...] = dis * dis * h + b_ref[...]
    dis_ref[...] = dis


def _tc1(deg2, x_p, W1, b1r):
    return pl.pallas_call(
        _tc1_body,
        grid=(_GRID,),
        in_specs=[
            pl.BlockSpec((_RB, 1), lambda i: (i, 0)),
            pl.BlockSpec((_RB, D), lambda i: (i, 0)),
            pl.BlockSpec((D, D), lambda i: (0, 0)),
            pl.BlockSpec((1, D), lambda i: (0, 0)),
        ],
        out_specs=[
            pl.BlockSpec((_RB, D), lambda i: (i, 0)),
            pl.BlockSpec((_RB, D), lambda i: (i, 0)),
            pl.BlockSpec((_RB, 1), lambda i: (i, 0)),
        ],
        out_shape=[
            jax.ShapeDtypeStruct((N_PAD, D), jnp.float32),
            jax.ShapeDtypeStruct((N_PAD, D), jnp.float32),
            jax.ShapeDtypeStruct((N_PAD, 1), jnp.float32),
        ],
    )(deg2, x_p, W1, b1r)


def _tc2_body(s_ref, base_ref, dis_ref, w_ref, b_ref, u_ref, base2_ref):
    dis = dis_ref[...]
    h1 = jnp.maximum(dis * s_ref[...] + base_ref[...], 0.0)
    h2 = jnp.dot(h1, w_ref[...], preferred_element_type=jnp.float32)
    u_ref[...] = dis * h2
    base2_ref[...] = dis * dis * h2 + b_ref[...]


def _tc2(s1, base1, dis, W2, b2r):
    return pl.pallas_call(
        _tc2_body,
        grid=(_GRID,),
        in_specs=[
            pl.BlockSpec((_RB, D), lambda i: (i, 0)),
            pl.BlockSpec((_RB, D), lambda i: (i, 0)),
            pl.BlockSpec((_RB, 1), lambda i: (i, 0)),
            pl.BlockSpec((D, D), lambda i: (0, 0)),
            pl.BlockSpec((1, D), lambda i: (0, 0)),
        ],
        out_specs=[
            pl.BlockSpec((_RB, D), lambda i: (i, 0)),
            pl.BlockSpec((_RB, D), lambda i: (i, 0)),
        ],
        out_shape=[
            jax.ShapeDtypeStruct((N_PAD, D), jnp.float32),
            jax.ShapeDtypeStruct((N_PAD, D), jnp.float32),
        ],
    )(s1, base1, dis, W2, b2r)


def _tc3_body(s_ref, base_ref, dis_ref, z_ref):
    z_ref[...] = dis_ref[...] * s_ref[...] + base_ref[...]


def _tc3(s2, base2, dis):
    return pl.pallas_call(
        _tc3_body,
        grid=(_GRID,),
        in_specs=[
            pl.BlockSpec((_RB, D), lambda i: (i, 0)),
            pl.BlockSpec((_RB, D), lambda i: (i, 0)),
            pl.BlockSpec((_RB, 1), lambda i: (i, 0)),
        ],
        out_specs=pl.BlockSpec((_RB, D), lambda i: (i, 0)),
        out_shape=jax.ShapeDtypeStruct((N_PAD, D), jnp.float32),
    )(s2, base2, dis)


@jax.jit
def kernel(x, edge_index, W1, b1, W2, b2):
    ei = edge_index.astype(jnp.int32)
    src = jnp.concatenate([ei[0], jnp.zeros((E_PAD - E,), jnp.int32)])
    dst = jnp.concatenate([ei[1], jnp.full((E_PAD - E,), N, jnp.int32)])
    x_p = jnp.pad(x, ((0, N_PAD - N), (0, 0)))

    lists, counts, deg = _route_call(src, dst)
    u1, base1, dis = _tc1(deg.reshape(N_PAD, 1), x_p, W1, b1.reshape(1, D))
    s1 = _segsum_call(u1, lists, counts)
    u2, base2 = _tc2(s1, base1, dis, W2, b2.reshape(1, D))
    s2 = _segsum_call(u2, lists, counts)
    z = _tc3(s2, base2, dis)
    logits = _decode_call(z, src, dst)
    return logits[:E]


# speedup vs baseline: 1.0337x; 1.0337x over previous
import jax
import jax.numpy as jnp
from jax import lax
from jax.experimental import pallas as pl
from jax.experimental.pallas import tpu as pltpu
from jax.experimental.pallas import tpu_sc as plsc

N = 10000
E = 160000
D = 256

NC = 2
NS = 16
NW = NC * NS
L = 16

N_PAD = 10240
E_PAD = 163840
CHUNK = 128

WIN = N_PAD // NW
TRASHW = WIN
WROWS = WIN + 8

BLK = 2048
NBLK = E_PAD // BLK
SPILL = 8192
BUFN = 10496
TAIL = BUFN - SPILL
LIST_CAP = E_PAD + 128
SHIFT = 9

EPW = E_PAD // NW
DGROUPS = EPW // CHUNK

_MESH = plsc.VectorSubcoreMesh(
    core_axis_name="c", subcore_axis_name="s", num_cores=NC, num_subcores=NS
)
_SC_PARAMS = pltpu.CompilerParams(needs_layout_passes=False)


def _wid():
    return lax.axis_index("s") * NC + lax.axis_index("c")


def _route_body(src_hbm, dst_hbm, lists_hbm, counts_hbm, deg_hbm,
                buf, sidc, dlc, degf, cntv):
    wid = _wid()
    lo = wid * WIN
    base = wid * LIST_CAP
    ones = jnp.ones((L,), jnp.float32)
    zf = jnp.zeros((L,), jnp.float32)

    def zb(i, _):
        degf[pl.ds(i * L, L)] = zf
        return 0

    lax.fori_loop(0, (WIN + L) // L, zb, 0)

    def blk_body(b, carry):
        nm, oo = carry
        off = b * BLK
        pltpu.sync_copy(src_hbm.at[pl.ds(pl.multiple_of(off, 128), BLK)], sidc)
        pltpu.sync_copy(dst_hbm.at[pl.ds(pl.multiple_of(off, 128), BLK)], dlc)

        def g_body(gq, nm):
            svals, cnts = [], []
            for t in range(8):
                g = gq * 8 + t
                sv = sidc[pl.ds(g * L, L)]
                dv = dlc[pl.ds(g * L, L)]
                m = (dv >= lo) & (dv < lo + WIN)
                dl = jnp.where(m, dv - lo, jnp.int32(TRASHW))
                plsc.addupdate_scatter(degf, [dl], ones)
                packed = (sv << SHIFT) | dl
                key = jnp.where(m, jnp.int32(0), jnp.int32(1))
                _, sval = plsc.sort_key_val(key, packed)
                svals.append(sval)
                cnts.append(plsc.all_reduce_population_count(m)[0])
            off = nm
            for t in range(8):
                buf[pl.ds(off, L)] = svals[t]
                off = off + cnts[t]
            return off

        nm = lax.fori_loop(0, BLK // L // 8, g_body, nm)

        do_spill = nm >= SPILL

        @pl.when(do_spill)
        def _():
            pltpu.sync_copy(
                buf.at[pl.ds(0, SPILL)],
                lists_hbm.at[pl.ds(pl.multiple_of(base + oo, 128), SPILL)]
            )

            def mv(q, _):
                buf[pl.ds(q * L, L)] = buf[pl.ds(SPILL + q * L, L)]
                return 0

            lax.fori_loop(0, TAIL // L, mv, 0)

        nm = jnp.where(do_spill, nm - SPILL, nm)
        oo = jnp.where(do_spill, oo + SPILL, oo)
        return (nm, oo)

    nm, oo = lax.fori_loop(0, NBLK, blk_body, (jnp.int32(0), jnp.int32(0)))

    trash16 = jnp.full((L,), TRASHW, jnp.int32)

    def pad_b(j, _):
        buf[pl.ds(nm + j * L, L)] = trash16
        return 0

    lax.fori_loop(0, CHUNK // L, pad_b, 0)

    nflush = (nm + CHUNK - 1) // CHUNK

    def fl(q, _):
        pltpu.sync_copy(
            buf.at[pl.ds(q * CHUNK, CHUNK)],
            lists_hbm.at[pl.ds(pl.multiple_of(base + oo + q * CHUNK, 128), CHUNK)],
        )
        return 0

    lax.fori_loop(0, nflush, fl, 0)

    cntv[pl.ds(0, L)] = jnp.full((L,), 0, jnp.int32) + (oo + nm)
    pltpu.sync_copy(cntv, counts_hbm.at[pl.ds(pl.multiple_of(wid * L, 16), L)])
    pltpu.sync_copy(degf.at[pl.ds(0, WIN)],
                    deg_hbm.at[pl.ds(pl.multiple_of(wid * WIN, 64), WIN)])


_route_call = pl.kernel(
    _route_body,
    out_type=(
        jax.ShapeDtypeStruct((NW * LIST_CAP,), jnp.int32),
        jax.ShapeDtypeStruct((NW * L,), jnp.int32),
        jax.ShapeDtypeStruct((N_PAD,), jnp.float32),
    ),
    mesh=_MESH,
    compiler_params=_SC_PARAMS,
    scratch_types=[
        pltpu.VMEM((BUFN,), jnp.int32),
        pltpu.VMEM((BLK,), jnp.int32),
        pltpu.VMEM((BLK,), jnp.int32),
        pltpu.VMEM((WIN + L,), jnp.float32),
        pltpu.VMEM((L,), jnp.int32),
    ],
)


CH = 64
MASK9 = (1 << SHIFT) - 1


def _segsum_body(u_hbm, lists_hbm, counts_hbm, out_hbm,
                 win, pk, sidc0, sidc1, dlc0, dlc1, stg0, stg1, cntv,
                 sem0, sem1):
    wid = _wid()
    base = wid * LIST_CAP
    zf = jnp.zeros((L,), jnp.float32)
    sidc = (sidc0, sidc1)
    dlc = (dlc0, dlc1)
    stg = (stg0, stg1)
    sem = (sem0, sem1)

    def zb(i, _):
        for j in range(D // L):
            win[i, pl.ds(j * L, L)] = zf
        return 0

    lax.fori_loop(0, WROWS, zb, 0)

    pltpu.sync_copy(counts_hbm.at[pl.ds(pl.multiple_of(wid * L, 16), L)], cntv)
    total = cntv[pl.ds(0, L)][0]
    nch = (total + CH - 1) // CH

    def start(ci, b):
        pltpu.sync_copy(
            lists_hbm.at[pl.ds(pl.multiple_of(base + ci * CH, 64), CH)], pk)
        for j in range(CH // L):
            p = pk[pl.ds(j * L, L)]
            sidc[b][pl.ds(j * L, L)] = p >> SHIFT
            dlc[b][pl.ds(j * L, L)] = p & MASK9
        pltpu.async_copy(u_hbm.at[sidc[b]], stg[b], sem[b])

    @pl.when(nch > 0)
    def _():
        start(0, 0)

    def outer(t, _):
        for b in range(2):
            ci = t * 2 + b

            @pl.when(ci < nch)
            def _(b=b, ci=ci):
                pltpu.make_async_copy(u_hbm.at[sidc[b]], stg[b], sem[b]).wait()

                @pl.when(ci + 1 < nch)
                def _():
                    start(ci + 1, 1 - b)

                def accg(g, _):
                    rv = dlc[b][pl.ds(g * L, L)]
                    for i in range(L):
                        k = g * L + i
                        r = rv[i]
                        for j in range(D // L):
                            plsc.addupdate(
                                win.at[r, pl.ds(j * L, L)],
                                stg[b][k, pl.ds(j * L, L)],
                            )
                    return 0

                lax.fori_loop(0, CH // L, accg, 0)

        return 0

    lax.fori_loop(0, (nch + 1) // 2, outer, 0)

    pltpu.sync_copy(win.at[pl.ds(0, WIN)],
                    out_hbm.at[pl.ds(pl.multiple_of(wid * WIN, 64), WIN)])


_segsum_call = pl.kernel(
    _segsum_body,
    out_type=jax.ShapeDtypeStruct((N_PAD, D), jnp.float32),
    mesh=_MESH,
    compiler_params=_SC_PARAMS,
    scratch_types=[
        pltpu.VMEM((WROWS, D), jnp.float32),
        pltpu.VMEM((CH,), jnp.int32),
        pltpu.VMEM((CH,), jnp.int32),
        pltpu.VMEM((CH,), jnp.int32),
        pltpu.VMEM((CH + L,), jnp.int32),
        pltpu.VMEM((CH + L,), jnp.int32),
        pltpu.VMEM((CH, D), jnp.float32),
        pltpu.VMEM((CH, D), jnp.float32),
        pltpu.VMEM((L,), jnp.int32),
        pltpu.SemaphoreType.DMA,
        pltpu.SemaphoreType.DMA,
    ],
)


DCH = 64
DCHUNKS = EPW // DCH


def _decode_body(z_hbm, src_hbm, dst_hbm, out_hbm,
                 sidx0, sidx1, didx0, didx1, ar0, ar1, br0, br1, stag, outv,
                 sa0, sa1, sb0, sb1):
    wid = _wid()
    iota = lax.iota(jnp.int32, L)
    sidx = (sidx0, sidx1)
    didx = (didx0, didx1)
    ar = (ar0, ar1)
    br = (br0, br1)
    sa = (sa0, sa1)
    sb = (sb0, sb1)

    def start(ci, b):
        off = pl.multiple_of(wid * EPW + ci * DCH, 64)
        pltpu.sync_copy(src_hbm.at[pl.ds(off, DCH)], sidx[b])
        pltpu.sync_copy(dst_hbm.at[pl.ds(off, DCH)], didx[b])
        pltpu.async_copy(z_hbm.at[sidx[b]], ar[b], sa[b])
        pltpu.async_copy(z_hbm.at[didx[b]], br[b], sb[b])

    start(0, 0)

    def outer(t, _):
        for b in range(2):
            ci = t * 2 + b
            pltpu.make_async_copy(z_hbm.at[sidx[b]], ar[b], sa[b]).wait()
            pltpu.make_async_copy(z_hbm.at[didx[b]], br[b], sb[b]).wait()

            @pl.when(ci + 1 < DCHUNKS)
            def _(b=b, ci=ci):
                start(ci + 1, 1 - b)

            def group(g, _, b=b):
                for i in range(L):
                    r = g * L + i
                    acc0 = ar[b][r, pl.ds(0, L)] * br[b][r, pl.ds(0, L)]
                    acc1 = ar[b][r, pl.ds(L, L)] * br[b][r, pl.ds(L, L)]
                    acc2 = ar[b][r, pl.ds(2 * L, L)] * br[b][r, pl.ds(2 * L, L)]
                    acc3 = ar[b][r, pl.ds(3 * L, L)] * br[b][r, pl.ds(3 * L, L)]
                    for j in range(4, D // L, 4):
                        acc0 = acc0 + ar[b][r, pl.ds(j * L, L)] * br[b][r, pl.ds(j * L, L)]
                        acc1 = acc1 + ar[b][r, pl.ds((j + 1) * L, L)] * br[b][r, pl.ds((j + 1) * L, L)]
                        acc2 = acc2 + ar[b][r, pl.ds((j + 2) * L, L)] * br[b][r, pl.ds((j + 2) * L, L)]
                        acc3 = acc3 + ar[b][r, pl.ds((j + 3) * L, L)] * br[b][r, pl.ds((j + 3) * L, L)]
                    stag[i, pl.ds(0, L)] = (acc0 + acc1) + (acc2 + acc3)
                out16 = jnp.zeros((L,), jnp.float32)
                for k in range(L):
                    col = jnp.full((L,), 0, jnp.int32) + k
                    out16 = out16 + plsc.load_gather(stag, [iota, col])
                outv[pl.ds(g * L, L)] = out16
                return 0

            lax.fori_loop(0, DCH // L, group, 0)
            off = pl.multiple_of(wid * EPW + ci * DCH, 64)
            pltpu.sync_copy(outv, out_hbm.at[pl.ds(off, DCH)])

        return 0

    lax.fori_loop(0, DCHUNKS // 2, outer, 0)


_decode_call = pl.kernel(
    _decode_body,
    out_type=jax.ShapeDtypeStruct((E_PAD,), jnp.float32),
    mesh=_MESH,
    compiler_params=_SC_PARAMS,
    scratch_types=[
        pltpu.VMEM((DCH,), jnp.int32),
        pltpu.VMEM((DCH,), jnp.int32),
        pltpu.VMEM((DCH,), jnp.int32),
        pltpu.VMEM((DCH,), jnp.int32),
        pltpu.VMEM((DCH, D), jnp.float32),
        pltpu.VMEM((DCH, D), jnp.float32),
        pltpu.VMEM((DCH, D), jnp.float32),
        pltpu.VMEM((DCH, D), jnp.float32),
        pltpu.VMEM((L, L), jnp.float32),
        pltpu.VMEM((DCH,), jnp.float32),
        pltpu.SemaphoreType.DMA,
        pltpu.SemaphoreType.DMA,
        pltpu.SemaphoreType.DMA,
        pltpu.SemaphoreType.DMA,
    ],
)


_RB = 1024
_GRID = N_PAD // _RB


def _tc1_body(deg_ref, x_ref, w_ref, b_ref, u_ref, base_ref, dis_ref):
    dis = lax.rsqrt(deg_ref[...] + 1.0)
    h = jnp.dot(x_ref[...], w_ref[...], preferred_element_type=jnp.float32)
    u_ref[...] = dis * h
    base_ref[...] = dis * dis * h + b_ref[...]
    dis_ref[...] = dis


def _tc1(deg2, x_p, W1, b1r):
    return pl.pallas_call(
        _tc1_body,
        grid=(_GRID,),
        in_specs=[
            pl.BlockSpec((_RB, 1), lambda i: (i, 0)),
            pl.BlockSpec((_RB, D), lambda i: (i, 0)),
            pl.BlockSpec((D, D), lambda i: (0, 0)),
            pl.BlockSpec((1, D), lambda i: (0, 0)),
        ],
        out_specs=[
            pl.BlockSpec((_RB, D), lambda i: (i, 0)),
            pl.BlockSpec((_RB, D), lambda i: (i, 0)),
            pl.BlockSpec((_RB, 1), lambda i: (i, 0)),
        ],
        out_shape=[
            jax.ShapeDtypeStruct((N_PAD, D), jnp.float32),
            jax.ShapeDtypeStruct((N_PAD, D), jnp.float32),
            jax.ShapeDtypeStruct((N_PAD, 1), jnp.float32),
        ],
    )(deg2, x_p, W1, b1r)


def _tc2_body(s_ref, base_ref, dis_ref, w_ref, b_ref, u_ref, base2_ref):
    dis = dis_ref[...]
    h1 = jnp.maximum(dis * s_ref[...] + base_ref[...], 0.0)
    h2 = jnp.dot(h1, w_ref[...], preferred_element_type=jnp.float32)
    u_ref[...] = dis * h2
    base2_ref[...] = dis * dis * h2 + b_ref[...]


def _tc2(s1, base1, dis, W2, b2r):
    return pl.pallas_call(
        _tc2_body,
        grid=(_GRID,),
        in_specs=[
            pl.BlockSpec((_RB, D), lambda i: (i, 0)),
            pl.BlockSpec((_RB, D), lambda i: (i, 0)),
            pl.BlockSpec((_RB, 1), lambda i: (i, 0)),
            pl.BlockSpec((D, D), lambda i: (0, 0)),
            pl.BlockSpec((1, D), lambda i: (0, 0)),
        ],
        out_specs=[
            pl.BlockSpec((_RB, D), lambda i: (i, 0)),
            pl.BlockSpec((_RB, D), lambda i: (i, 0)),
        ],
        out_shape=[
            jax.ShapeDtypeStruct((N_PAD, D), jnp.float32),
            jax.ShapeDtypeStruct((N_PAD, D), jnp.float32),
        ],
    )(s1, base1, dis, W2, b2r)


def _tc3_body(s_ref, base_ref, dis_ref, z_ref):
    z_ref[...] = dis_ref[...] * s_ref[...] + base_ref[...]


def _tc3(s2, base2, dis):
    return pl.pallas_call(
        _tc3_body,
        grid=(_GRID,),
        in_specs=[
            pl.BlockSpec((_RB, D), lambda i: (i, 0)),
            pl.BlockSpec((_RB, D), lambda i: (i, 0)),
            pl.BlockSpec((_RB, 1), lambda i: (i, 0)),
        ],
        out_specs=pl.BlockSpec((_RB, D), lambda i: (i, 0)),
        out_shape=jax.ShapeDtypeStruct((N_PAD, D), jnp.float32),
    )(s2, base2, dis)


@jax.jit
def kernel(x, edge_index, W1, b1, W2, b2):
    ei = edge_index.astype(jnp.int32)
    src = jnp.concatenate([ei[0], jnp.zeros((E_PAD - E,), jnp.int32)])
    dst = jnp.concatenate([ei[1], jnp.full((E_PAD - E,), N, jnp.int32)])
    x_p = jnp.pad(x, ((0, N_PAD - N), (0, 0)))

    lists, counts, deg = _route_call(src, dst)
    u1, base1, dis = _tc1(deg.reshape(N_PAD, 1), x_p, W1, b1.reshape(1, D))
    s1 = _segsum_call(u1, lists, counts)
    u2, base2 = _tc2(s1, base1, dis, W2, b2.reshape(1, D))
    s2 = _segsum_call(u2, lists, counts)
    z = _tc3(s2, base2, dis)
    logits = _decode_call(z, src, dst)
    return logits[:E]
